# baseline (speedup 1.0000x reference)
import jax
import jax.numpy as jnp
from jax import lax
from jax.experimental import pallas as pl
from jax.experimental.pallas import tpu as pltpu
from jax.experimental.pallas import tpu_sc as plsc

_N = 10000
_E = 320000
_DF = 128
_H = 256
_HH = 128
_NC = 2
_NS = 16
_NW = _NC * _NS
_CH = 80
_RNG = 3456
_NPASS = 3
_AGGP = _RNG * _NPASS

_BN = 1000

_mesh = plsc.VectorSubcoreMesh(
    core_axis_name="c", subcore_axis_name="s", num_cores=_NC, num_subcores=_NS)


def _fill2d(ref, rows, width, value):
    v = jnp.full((16,), value, jnp.float32)

    def body(r, carry):
        for k in range(width // 16):
            ref[r, pl.ds(k * 16, 16)] = v
        return carry

    lax.fori_loop(0, rows, body, 0)


def _sc_degree(dst2d):
    def body(dst_ref, out_ref, idx_v, acc1d):
        c = lax.axis_index("c")
        s = lax.axis_index("s")
        w = s * _NC + c
        ones = jnp.ones((16,), jnp.float32)
        zeros = jnp.zeros((16,), jnp.float32)

        def zrow(r, carry):
            acc1d[pl.ds(r * 16, 16)] = zeros
            return carry

        lax.fori_loop(0, _N // 16, zrow, 0)
        pltpu.sync_copy(dst_ref.at[w], idx_v)

        def chunk(j, carry):
            for k in range(_CH // 16):
                idx = idx_v[j, pl.ds(k * 16, 16)]
                plsc.addupdate_scatter(acc1d, [idx], ones)
            return carry

        lax.fori_loop(0, 125, chunk, 0)
        pltpu.sync_copy(acc1d, out_ref.at[w])

    return pl.kernel(
        body,
        out_type=jax.ShapeDtypeStruct((_NW, _N), jnp.float32),
        mesh=_mesh,
        scratch_types=[
            pltpu.VMEM((125, _CH), jnp.int32),
            pltpu.VMEM((_N,), jnp.float32),
        ],
        compiler_params=pltpu.CompilerParams(needs_layout_passes=False),
    )(dst2d)


def _tc_dinv(partials):
    def body(p_ref, o_ref):
        o_ref[...] = lax.rsqrt(
            jnp.sum(p_ref[...], axis=0, keepdims=True) + 1.0)

    return pl.pallas_call(
        body,
        grid=(1,),
        in_specs=[pl.BlockSpec((_NW, _N), lambda i: (0, 0))],
        out_specs=pl.BlockSpec((1, _N), lambda i: (0, 0)),
        out_shape=jax.ShapeDtypeStruct((1, _N), jnp.float32),
    )(partials)


def _make_sc_aggregate():
    _NB = 2
    _IGN = -1

    def body(gA_ref, gB_ref, src_ref, dst_ref, outA, outB,
             idxS, idxD, idxS2, idxD2, rows, zbuf, acc, s0, s1):
        sems = (s0, s1)
        c = lax.axis_index("c")
        s = lax.axis_index("s")
        pltpu.sync_copy(src_ref.at[s], idxS)
        pltpu.sync_copy(dst_ref.at[s], idxD)
        _fill2d(zbuf, 8, _HH, 0.0)

        def remap(j, b, p):
            for k in range(_CH // 16):
                d = idxD[j, pl.ds(k * 16, 16)]
                sv = idxS[j, pl.ds(k * 16, 16)]
                loc = d - p * _RNG
                msk = (loc >= 0) & (loc < _RNG)
                idxD2[b, pl.ds(k * 16, 16)] = jnp.where(msk, loc, _IGN)
                idxS2[b, pl.ds(k * 16, 16)] = jnp.where(msk, sv, _IGN)

        def gidx(b):
            return plsc.Indices(idxS2.at[b], ignored_value=_IGN)

        def sidx(b):
            return plsc.Indices(idxD2.at[b], ignored_value=_IGN)

        for p in range(_NPASS):
            for i in range(27):
                pltpu.sync_copy(zbuf, acc.at[pl.ds(s * 216 + i * 8, 8)])
            plsc.subcore_barrier()

            def run(tbl):
                def _go():
                    for b in range(_NB):
                        remap(b, b, p)
                        pltpu.async_copy(tbl.at[gidx(b)], rows.at[b],
                                         sems[b])

                    def group(g, carry):
                        for b in range(_NB):
                            j = g * _NB + b
                            pltpu.make_async_copy(
                                tbl.at[gidx(b)], rows.at[b],
                                sems[b]).wait()
                            pltpu.sync_copy(rows.at[b],
                                            acc.at[sidx(b)], add=True)

                            @pl.when(g < 250 // _NB - 1)
                            def _():
                                remap(j + _NB, b, p)
                                pltpu.async_copy(tbl.at[gidx(b)],
                                                 rows.at[b], sems[b])
                        return carry

                    lax.fori_loop(0, 250 // _NB, group, 0)
                return _go

            pl.when(c == 0)(run(gA_ref))
            pl.when(c == 1)(run(gB_ref))
            plsc.subcore_barrier()

            def wout(out_ref):
                def _go():
                    pltpu.sync_copy(acc.at[pl.ds(s * 216, 216)],
                                    out_ref.at[pl.ds(p * _RNG + s * 216, 216)])
                return _go

            pl.when(c == 0)(wout(outA))
            pl.when(c == 1)(wout(outB))
            plsc.subcore_barrier()

    return pl.kernel(
        body,
        out_type=(jax.ShapeDtypeStruct((_AGGP, _HH), jnp.float32),
                  jax.ShapeDtypeStruct((_AGGP, _HH), jnp.float32)),
        mesh=_mesh,
        scratch_types=[
            pltpu.VMEM((250, _CH), jnp.int32),
            pltpu.VMEM((250, _CH), jnp.int32),
            pltpu.VMEM((_NB, _CH), jnp.int32),
            pltpu.VMEM((_NB, _CH), jnp.int32),
            pltpu.VMEM((_NB, _CH, _HH), jnp.float32),
            pltpu.VMEM((8, _HH), jnp.float32),
            pltpu.VMEM_SHARED((_RNG, _HH), jnp.float32),
            pltpu.SemaphoreType.DMA,
            pltpu.SemaphoreType.DMA,
        ],
    )


_sc_aggregate = _make_sc_aggregate()


def _sc_edge_sum(P1, P2, src2d, dst2d):
    def body(P1_ref, P2_ref, src_ref, dst_ref, m_out,
             idxS, idxD, bufA, bufB, sa0, sa1, sb0, sb1):
        sas = (sa0, sa1)
        sbs = (sb0, sb1)
        c = lax.axis_index("c")
        s = lax.axis_index("s")
        w = s * _NC + c
        pltpu.sync_copy(src_ref.at[w], idxS)
        pltpu.sync_copy(dst_ref.at[w], idxD)

        def start(j, b):
            pltpu.async_copy(P1_ref.at[idxS.at[j]], bufA.at[b], sas[b])
            pltpu.async_copy(P2_ref.at[idxD.at[j]], bufB.at[b], sbs[b])

        def finish(j, b):
            pltpu.make_async_copy(P1_ref.at[idxS.at[j]], bufA.at[b],
                                  sas[b]).wait()
            pltpu.make_async_copy(P2_ref.at[idxD.at[j]], bufB.at[b],
                                  sbs[b]).wait()

            def addrow(r, carry2):
                for k in range(_H // 16):
                    plsc.addupdate(bufA.at[b, r, pl.ds(k * 16, 16)],
                                   bufB[b, r, pl.ds(k * 16, 16)])
                return carry2

            lax.fori_loop(0, _CH, addrow, 0)
            pltpu.sync_copy(bufA.at[b],
                            m_out.at[pl.ds(w * 10000 + j * _CH, _CH)])

        for b in range(2):
            start(b, b)

        def group(g, carry):
            j0 = g * 2
            finish(j0, 0)
            start(j0 + 2, 0)
            finish(j0 + 1, 1)

            @pl.when(g < 61)
            def _():
                start(j0 + 3, 1)
            return carry

        lax.fori_loop(0, 62, group, 0)
        finish(124, 0)

    return pl.kernel(
        body,
        out_type=jax.ShapeDtypeStruct((_E, _H), jnp.float32),
        mesh=_mesh,
        scratch_types=[
            pltpu.VMEM((125, _CH), jnp.int32),
            pltpu.VMEM((125, _CH), jnp.int32),
            pltpu.VMEM((2, _CH, _H), jnp.float32),
            pltpu.VMEM((2, _CH, _H), jnp.float32),
            pltpu.SemaphoreType.DMA,
            pltpu.SemaphoreType.DMA,
            pltpu.SemaphoreType.DMA,
            pltpu.SemaphoreType.DMA,
        ],
    )(P1, P2, src2d, dst2d)


def _tc_layer1(x, W1, b1r, dinv_col):
    def body(x_ref, w_ref, b_ref, d_ref, gA_ref, gB_ref):
        h = jnp.dot(x_ref[...], w_ref[...],
                    preferred_element_type=jnp.float32) + b_ref[...]
        g = h * d_ref[...]
        gA_ref[...] = g[:, :_HH]
        gB_ref[...] = g[:, _HH:]

    return pl.pallas_call(
        body,
        grid=(_N // _BN,),
        in_specs=[
            pl.BlockSpec((_BN, _DF), lambda i: (i, 0)),
            pl.BlockSpec((_DF, _H), lambda i: (0, 0)),
            pl.BlockSpec((1, _H), lambda i: (0, 0)),
            pl.BlockSpec((_BN, 1), lambda i: (i, 0)),
        ],
        out_specs=[pl.BlockSpec((_BN, _HH), lambda i: (i, 0))] * 2,
        out_shape=[jax.ShapeDtypeStruct((_N, _HH), jnp.float32)] * 2,
    )(x, W1, b1r, dinv_col)


def _tc_mid_a(aggA, aggB, gA, gB, dinv_col):
    def body(aA, aB, gAr, gBr, d_ref, h1_ref, s1_ref, s2_ref):
        i = pl.program_id(0)
        h = jnp.concatenate(
            [aA[...] + gAr[...], aB[...] + gBr[...]], axis=1) * d_ref[...]
        h1_ref[...] = h

        @pl.when(i == 0)
        def _():
            s1_ref[...] = jnp.zeros_like(s1_ref)
            s2_ref[...] = jnp.zeros_like(s2_ref)

        s1_ref[...] += jnp.sum(h, axis=0, keepdims=True)
        s2_ref[...] += jnp.sum(h * h, axis=0, keepdims=True)

    return pl.pallas_call(
        body,
        grid=(_N // _BN,),
        in_specs=[
            pl.BlockSpec((_BN, _HH), lambda i: (i, 0)),
            pl.BlockSpec((_BN, _HH), lambda i: (i, 0)),
            pl.BlockSpec((_BN, _HH), lambda i: (i, 0)),
            pl.BlockSpec((_BN, _HH), lambda i: (i, 0)),
            pl.BlockSpec((_BN, 1), lambda i: (i, 0)),
        ],
        out_specs=[
            pl.BlockSpec((_BN, _H), lambda i: (i, 0)),
            pl.BlockSpec((1, _H), lambda i: (0, 0)),
            pl.BlockSpec((1, _H), lambda i: (0, 0)),
        ],
        out_shape=[
            jax.ShapeDtypeStruct((_N, _H), jnp.float32),
            jax.ShapeDtypeStruct((1, _H), jnp.float32),
            jax.ShapeDtypeStruct((1, _H), jnp.float32),
        ],
    )(aggA, aggB, gA, gB, dinv_col)


def _tc_mid_b(h1, s1, s2, gammar, betar, W2, b2r, dinv_col):
    def body(h1_ref, s1_ref, s2_ref, gm, bt, w_ref, b_ref, d_ref,
             g2A_ref, g2B_ref):
        mu = s1_ref[...] / float(_N)
        var = s2_ref[...] / float(_N) - mu * mu
        hb = gm[...] * (h1_ref[...] - mu) * lax.rsqrt(var + 1e-5) + bt[...]
        hb = jnp.maximum(hb, 0.0)
        h2p = (jnp.dot(hb, w_ref[...], preferred_element_type=jnp.float32)
               + b_ref[...]) * d_ref[...]
        g2A_ref[...] = h2p[:, :_HH]
        g2B_ref[...] = h2p[:, _HH:]

    return pl.pallas_call(
        body,
        grid=(_N // _BN,),
        in_specs=[
            pl.BlockSpec((_BN, _H), lambda i: (i, 0)),
            pl.BlockSpec((1, _H), lambda i: (0, 0)),
            pl.BlockSpec((1, _H), lambda i: (0, 0)),
            pl.BlockSpec((1, _H), lambda i: (0, 0)),
            pl.BlockSpec((1, _H), lambda i: (0, 0)),
            pl.BlockSpec((_H, _H), lambda i: (0, 0)),
            pl.BlockSpec((1, _H), lambda i: (0, 0)),
            pl.BlockSpec((_BN, 1), lambda i: (i, 0)),
        ],
        out_specs=[pl.BlockSpec((_BN, _HH), lambda i: (i, 0))] * 2,
        out_shape=[jax.ShapeDtypeStruct((_N, _HH), jnp.float32)] * 2,
    )(h1, s1, s2, gammar, betar, W2, b2r, dinv_col)


def _tc_final_nodes(agg2A, agg2B, g2A, g2B, dinv_col, Wm1, bm1r):
    def body(aA, aB, gAr, gBr, d_ref, w_ref, b_ref, p1_ref, p2_ref):
        h2 = jnp.concatenate(
            [aA[...] + gAr[...], aB[...] + gBr[...]], axis=1) * d_ref[...]
        h2 = jnp.maximum(h2, 0.0)
        p1_ref[...] = jnp.dot(h2, w_ref[:_H, :],
                              preferred_element_type=jnp.float32) + b_ref[...]
        p2_ref[...] = jnp.dot(h2, w_ref[_H:, :],
                              preferred_element_type=jnp.float32)

    return pl.pallas_call(
        body,
        grid=(_N // _BN,),
        in_specs=[
            pl.BlockSpec((_BN, _HH), lambda i: (i, 0)),
            pl.BlockSpec((_BN, _HH), lambda i: (i, 0)),
            pl.BlockSpec((_BN, _HH), lambda i: (i, 0)),
            pl.BlockSpec((_BN, _HH), lambda i: (i, 0)),
            pl.BlockSpec((_BN, 1), lambda i: (i, 0)),
            pl.BlockSpec((2 * _H, _H), lambda i: (0, 0)),
            pl.BlockSpec((1, _H), lambda i: (0, 0)),
        ],
        out_specs=[pl.BlockSpec((_BN, _H), lambda i: (i, 0))] * 2,
        out_shape=[jax.ShapeDtypeStruct((_N, _H), jnp.float32)] * 2,
    )(agg2A, agg2B, g2A, g2B, dinv_col, Wm1, bm1r)


_BE = 2000


def _tc_classifier(m, Wm2, bm2r):
    def body(m_ref, w_ref, b_ref, o_ref):
        v = jnp.dot(jnp.maximum(m_ref[...], 0.0), w_ref[...],
                    preferred_element_type=jnp.float32) + b_ref[...]
        o_ref[...] = jax.nn.sigmoid(v)

    return pl.pallas_call(
        body,
        grid=(_E // _BE,),
        in_specs=[
            pl.BlockSpec((_BE, _H), lambda i: (i, 0)),
            pl.BlockSpec((_H, 1), lambda i: (0, 0)),
            pl.BlockSpec((1, 1), lambda i: (0, 0)),
        ],
        out_specs=pl.BlockSpec((_BE, 1), lambda i: (i, 0)),
        out_shape=jax.ShapeDtypeStruct((_E, 1), jnp.float32),
    )(m, Wm2, bm2r)


def kernel(x, edge_index, W1, b1, gamma, beta, W2, b2, Wm1, bm1, Wm2, bm2):
    src16 = edge_index[0].reshape(_NS, _E // (_NS * _CH), _CH)
    dst16 = edge_index[1].reshape(_NS, _E // (_NS * _CH), _CH)
    src32 = edge_index[0].reshape(_NW, _E // (_NW * _CH), _CH)
    dst32 = edge_index[1].reshape(_NW, _E // (_NW * _CH), _CH)

    partials = _sc_degree(dst32)
    dinv_col = _tc_dinv(partials).reshape(_N, 1)
    gA, gB = _tc_layer1(x, W1, b1.reshape(1, _H), dinv_col)
    aggA, aggB = _sc_aggregate(gA, gB, src16, dst16)
    h1, s1, s2 = _tc_mid_a(aggA, aggB, gA, gB, dinv_col)
    g2A, g2B = _tc_mid_b(h1, s1, s2, gamma.reshape(1, _H), beta.reshape(1, _H),
                         W2, b2.reshape(1, _H), dinv_col)
    agg2A, agg2B = _sc_aggregate(g2A, g2B, src16, dst16)
    P1, P2 = _tc_final_nodes(agg2A, agg2B, g2A, g2B, dinv_col,
                             Wm1, bm1.reshape(1, _H))
    m = _sc_edge_sum(P1, P2, src32, dst32)
    return _tc_classifier(m, Wm2, bm2.reshape(1, 1))


# speedup vs baseline: 1.0006x; 1.0006x over previous
import jax
import jax.numpy as jnp
from jax import lax
from jax.experimental import pallas as pl
from jax.experimental.pallas import tpu as pltpu
from jax.experimental.pallas import tpu_sc as plsc

_N = 10000
_E = 320000
_DF = 128
_H = 256
_HH = 128
_NC = 2
_NS = 16
_NW = _NC * _NS
_CH = 80
_RNG = 3456
_NPASS = 3
_AGGP = _RNG * _NPASS

_BN = 1000

def _mesh():
    return plsc.VectorSubcoreMesh(
        core_axis_name="c", subcore_axis_name="s",
        num_cores=_NC, num_subcores=_NS)


def _fill2d(ref, rows, width, value):
    v = jnp.full((16,), value, jnp.float32)

    def body(r, carry):
        for k in range(width // 16):
            ref[r, pl.ds(k * 16, 16)] = v
        return carry

    lax.fori_loop(0, rows, body, 0)


def _sc_degree(dst2d):
    def body(dst_ref, out_ref, idx_v, acc1d):
        c = lax.axis_index("c")
        s = lax.axis_index("s")
        w = s * _NC + c
        ones = jnp.ones((16,), jnp.float32)
        zeros = jnp.zeros((16,), jnp.float32)

        def zrow(r, carry):
            acc1d[pl.ds(r * 16, 16)] = zeros
            return carry

        lax.fori_loop(0, _N // 16, zrow, 0)
        pltpu.sync_copy(dst_ref.at[w], idx_v)

        def chunk(j, carry):
            for k in range(_CH // 16):
                idx = idx_v[j, pl.ds(k * 16, 16)]
                plsc.addupdate_scatter(acc1d, [idx], ones)
            return carry

        lax.fori_loop(0, 125, chunk, 0)
        pltpu.sync_copy(acc1d, out_ref.at[w])

    return pl.kernel(
        body,
        out_type=jax.ShapeDtypeStruct((_NW, _N), jnp.float32),
        mesh=_mesh(),
        scratch_types=[
            pltpu.VMEM((125, _CH), jnp.int32),
            pltpu.VMEM((_N,), jnp.float32),
        ],
        compiler_params=pltpu.CompilerParams(needs_layout_passes=False),
    )(dst2d)


def _tc_dinv(partials):
    def body(p_ref, o_ref):
        o_ref[...] = lax.rsqrt(
            jnp.sum(p_ref[...], axis=0, keepdims=True) + 1.0)

    return pl.pallas_call(
        body,
        grid=(1,),
        in_specs=[pl.BlockSpec((_NW, _N), lambda i: (0, 0))],
        out_specs=pl.BlockSpec((1, _N), lambda i: (0, 0)),
        out_shape=jax.ShapeDtypeStruct((1, _N), jnp.float32),
    )(partials)


def _make_sc_aggregate():
    _NB = 2
    _IGN = -1

    def body(gA_ref, gB_ref, src_ref, dst_ref, outA, outB,
             idxS, idxD, idxS2, idxD2, rows, zbuf, acc, s0, s1):
        sems = (s0, s1)
        c = lax.axis_index("c")
        s = lax.axis_index("s")
        pltpu.sync_copy(src_ref.at[s], idxS)
        pltpu.sync_copy(dst_ref.at[s], idxD)
        _fill2d(zbuf, 8, _HH, 0.0)

        def remap(j, b, p):
            for k in range(_CH // 16):
                d = idxD[j, pl.ds(k * 16, 16)]
                sv = idxS[j, pl.ds(k * 16, 16)]
                loc = d - p * _RNG
                msk = (loc >= 0) & (loc < _RNG)
                idxD2[b, pl.ds(k * 16, 16)] = jnp.where(msk, loc, _IGN)
                idxS2[b, pl.ds(k * 16, 16)] = jnp.where(msk, sv, _IGN)

        def gidx(b):
            return plsc.Indices(idxS2.at[b], ignored_value=_IGN)

        def sidx(b):
            return plsc.Indices(idxD2.at[b], ignored_value=_IGN)

        for p in range(_NPASS):
            for i in range(27):
                pltpu.sync_copy(zbuf, acc.at[pl.ds(s * 216 + i * 8, 8)])
            plsc.subcore_barrier()

            def run(tbl):
                def _go():
                    for b in range(_NB):
                        remap(b, b, p)
                        pltpu.async_copy(tbl.at[gidx(b)], rows.at[b],
                                         sems[b])

                    def group(g, carry):
                        for b in range(_NB):
                            j = g * _NB + b
                            pltpu.make_async_copy(
                                tbl.at[gidx(b)], rows.at[b],
                                sems[b]).wait()
                            pltpu.sync_copy(rows.at[b],
                                            acc.at[sidx(b)], add=True)

                            @pl.when(g < 250 // _NB - 1)
                            def _():
                                remap(j + _NB, b, p)
                                pltpu.async_copy(tbl.at[gidx(b)],
                                                 rows.at[b], sems[b])
                        return carry

                    lax.fori_loop(0, 250 // _NB, group, 0)
                return _go

            pl.when(c == 0)(run(gA_ref))
            pl.when(c == 1)(run(gB_ref))
            plsc.subcore_barrier()

            def wout(out_ref):
                def _go():
                    pltpu.sync_copy(acc.at[pl.ds(s * 216, 216)],
                                    out_ref.at[pl.ds(p * _RNG + s * 216, 216)])
                return _go

            pl.when(c == 0)(wout(outA))
            pl.when(c == 1)(wout(outB))
            plsc.subcore_barrier()

    return pl.kernel(
        body,
        out_type=(jax.ShapeDtypeStruct((_AGGP, _HH), jnp.float32),
                  jax.ShapeDtypeStruct((_AGGP, _HH), jnp.float32)),
        mesh=_mesh(),
        scratch_types=[
            pltpu.VMEM((250, _CH), jnp.int32),
            pltpu.VMEM((250, _CH), jnp.int32),
            pltpu.VMEM((_NB, _CH), jnp.int32),
            pltpu.VMEM((_NB, _CH), jnp.int32),
            pltpu.VMEM((_NB, _CH, _HH), jnp.float32),
            pltpu.VMEM((8, _HH), jnp.float32),
            pltpu.VMEM_SHARED((_RNG, _HH), jnp.float32),
            pltpu.SemaphoreType.DMA,
            pltpu.SemaphoreType.DMA,
        ],
    )


_agg_impl = None


def _sc_aggregate(gA, gB, src2d, dst2d):
    global _agg_impl
    if _agg_impl is None:
        _agg_impl = _make_sc_aggregate()
    return _agg_impl(gA, gB, src2d, dst2d)


def _sc_edge_sum(P1, P2, src2d, dst2d):
    def body(P1_ref, P2_ref, src_ref, dst_ref, m_out,
             idxS, idxD, bufA, bufB, sa0, sa1, sb0, sb1):
        sas = (sa0, sa1)
        sbs = (sb0, sb1)
        c = lax.axis_index("c")
        s = lax.axis_index("s")
        w = s * _NC + c
        pltpu.sync_copy(src_ref.at[w], idxS)
        pltpu.sync_copy(dst_ref.at[w], idxD)

        def start(j, b):
            pltpu.async_copy(P1_ref.at[idxS.at[j]], bufA.at[b], sas[b])
            pltpu.async_copy(P2_ref.at[idxD.at[j]], bufB.at[b], sbs[b])

        def finish(j, b):
            pltpu.make_async_copy(P1_ref.at[idxS.at[j]], bufA.at[b],
                                  sas[b]).wait()
            pltpu.make_async_copy(P2_ref.at[idxD.at[j]], bufB.at[b],
                                  sbs[b]).wait()

            def addrow(r, carry2):
                for k in range(_H // 16):
                    plsc.addupdate(bufA.at[b, r, pl.ds(k * 16, 16)],
                                   bufB[b, r, pl.ds(k * 16, 16)])
                return carry2

            lax.fori_loop(0, _CH, addrow, 0)
            pltpu.sync_copy(bufA.at[b],
                            m_out.at[pl.ds(w * 10000 + j * _CH, _CH)])

        for b in range(2):
            start(b, b)

        def group(g, carry):
            j0 = g * 2
            finish(j0, 0)
            start(j0 + 2, 0)
            finish(j0 + 1, 1)

            @pl.when(g < 61)
            def _():
                start(j0 + 3, 1)
            return carry

        lax.fori_loop(0, 62, group, 0)
        finish(124, 0)

    return pl.kernel(
        body,
        out_type=jax.ShapeDtypeStruct((_E, _H), jnp.float32),
        mesh=_mesh(),
        scratch_types=[
            pltpu.VMEM((125, _CH), jnp.int32),
            pltpu.VMEM((125, _CH), jnp.int32),
            pltpu.VMEM((2, _CH, _H), jnp.float32),
            pltpu.VMEM((2, _CH, _H), jnp.float32),
            pltpu.SemaphoreType.DMA,
            pltpu.SemaphoreType.DMA,
            pltpu.SemaphoreType.DMA,
            pltpu.SemaphoreType.DMA,
        ],
    )(P1, P2, src2d, dst2d)


def _tc_layer1(x, W1, b1r, dinv_col):
    def body(x_ref, w_ref, b_ref, d_ref, gA_ref, gB_ref):
        h = jnp.dot(x_ref[...], w_ref[...],
                    preferred_element_type=jnp.float32) + b_ref[...]
        g = h * d_ref[...]
        gA_ref[...] = g[:, :_HH]
        gB_ref[...] = g[:, _HH:]

    return pl.pallas_call(
        body,
        grid=(_N // _BN,),
        in_specs=[
            pl.BlockSpec((_BN, _DF), lambda i: (i, 0)),
            pl.BlockSpec((_DF, _H), lambda i: (0, 0)),
            pl.BlockSpec((1, _H), lambda i: (0, 0)),
            pl.BlockSpec((_BN, 1), lambda i: (i, 0)),
        ],
        out_specs=[pl.BlockSpec((_BN, _HH), lambda i: (i, 0))] * 2,
        out_shape=[jax.ShapeDtypeStruct((_N, _HH), jnp.float32)] * 2,
    )(x, W1, b1r, dinv_col)


def _tc_mid_a(aggA, aggB, gA, gB, dinv_col):
    def body(aA, aB, gAr, gBr, d_ref, h1_ref, s1_ref, s2_ref):
        i = pl.program_id(0)
        h = jnp.concatenate(
            [aA[...] + gAr[...], aB[...] + gBr[...]], axis=1) * d_ref[...]
        h1_ref[...] = h

        @pl.when(i == 0)
        def _():
            s1_ref[...] = jnp.zeros_like(s1_ref)
            s2_ref[...] = jnp.zeros_like(s2_ref)

        s1_ref[...] += jnp.sum(h, axis=0, keepdims=True)
        s2_ref[...] += jnp.sum(h * h, axis=0, keepdims=True)

    return pl.pallas_call(
        body,
        grid=(_N // _BN,),
        in_specs=[
            pl.BlockSpec((_BN, _HH), lambda i: (i, 0)),
            pl.BlockSpec((_BN, _HH), lambda i: (i, 0)),
            pl.BlockSpec((_BN, _HH), lambda i: (i, 0)),
            pl.BlockSpec((_BN, _HH), lambda i: (i, 0)),
            pl.BlockSpec((_BN, 1), lambda i: (i, 0)),
        ],
        out_specs=[
            pl.BlockSpec((_BN, _H), lambda i: (i, 0)),
            pl.BlockSpec((1, _H), lambda i: (0, 0)),
            pl.BlockSpec((1, _H), lambda i: (0, 0)),
        ],
        out_shape=[
            jax.ShapeDtypeStruct((_N, _H), jnp.float32),
            jax.ShapeDtypeStruct((1, _H), jnp.float32),
            jax.ShapeDtypeStruct((1, _H), jnp.float32),
        ],
    )(aggA, aggB, gA, gB, dinv_col)


def _tc_mid_b(h1, s1, s2, gammar, betar, W2, b2r, dinv_col):
    def body(h1_ref, s1_ref, s2_ref, gm, bt, w_ref, b_ref, d_ref,
             g2A_ref, g2B_ref):
        mu = s1_ref[...] / float(_N)
        var = s2_ref[...] / float(_N) - mu * mu
        hb = gm[...] * (h1_ref[...] - mu) * lax.rsqrt(var + 1e-5) + bt[...]
        hb = jnp.maximum(hb, 0.0)
        h2p = (jnp.dot(hb, w_ref[...], preferred_element_type=jnp.float32)
               + b_ref[...]) * d_ref[...]
        g2A_ref[...] = h2p[:, :_HH]
        g2B_ref[...] = h2p[:, _HH:]

    return pl.pallas_call(
        body,
        grid=(_N // _BN,),
        in_specs=[
            pl.BlockSpec((_BN, _H), lambda i: (i, 0)),
            pl.BlockSpec((1, _H), lambda i: (0, 0)),
            pl.BlockSpec((1, _H), lambda i: (0, 0)),
            pl.BlockSpec((1, _H), lambda i: (0, 0)),
            pl.BlockSpec((1, _H), lambda i: (0, 0)),
            pl.BlockSpec((_H, _H), lambda i: (0, 0)),
            pl.BlockSpec((1, _H), lambda i: (0, 0)),
            pl.BlockSpec((_BN, 1), lambda i: (i, 0)),
        ],
        out_specs=[pl.BlockSpec((_BN, _HH), lambda i: (i, 0))] * 2,
        out_shape=[jax.ShapeDtypeStruct((_N, _HH), jnp.float32)] * 2,
    )(h1, s1, s2, gammar, betar, W2, b2r, dinv_col)


def _tc_final_nodes(agg2A, agg2B, g2A, g2B, dinv_col, Wm1, bm1r):
    def body(aA, aB, gAr, gBr, d_ref, w_ref, b_ref, p1_ref, p2_ref):
        h2 = jnp.concatenate(
            [aA[...] + gAr[...], aB[...] + gBr[...]], axis=1) * d_ref[...]
        h2 = jnp.maximum(h2, 0.0)
        p1_ref[...] = jnp.dot(h2, w_ref[:_H, :],
                              preferred_element_type=jnp.float32) + b_ref[...]
        p2_ref[...] = jnp.dot(h2, w_ref[_H:, :],
                              preferred_element_type=jnp.float32)

    return pl.pallas_call(
        body,
        grid=(_N // _BN,),
        in_specs=[
            pl.BlockSpec((_BN, _HH), lambda i: (i, 0)),
            pl.BlockSpec((_BN, _HH), lambda i: (i, 0)),
            pl.BlockSpec((_BN, _HH), lambda i: (i, 0)),
            pl.BlockSpec((_BN, _HH), lambda i: (i, 0)),
            pl.BlockSpec((_BN, 1), lambda i: (i, 0)),
            pl.BlockSpec((2 * _H, _H), lambda i: (0, 0)),
            pl.BlockSpec((1, _H), lambda i: (0, 0)),
        ],
        out_specs=[pl.BlockSpec((_BN, _H), lambda i: (i, 0))] * 2,
        out_shape=[jax.ShapeDtypeStruct((_N, _H), jnp.float32)] * 2,
    )(agg2A, agg2B, g2A, g2B, dinv_col, Wm1, bm1r)


_BE = 2000


def _tc_classifier(m, Wm2, bm2r):
    def body(m_ref, w_ref, b_ref, o_ref):
        v = jnp.dot(jnp.maximum(m_ref[...], 0.0), w_ref[...],
                    preferred_element_type=jnp.float32) + b_ref[...]
        o_ref[...] = jax.nn.sigmoid(v)

    return pl.pallas_call(
        body,
        grid=(_E // _BE,),
        in_specs=[
            pl.BlockSpec((_BE, _H), lambda i: (i, 0)),
            pl.BlockSpec((_H, 1), lambda i: (0, 0)),
            pl.BlockSpec((1, 1), lambda i: (0, 0)),
        ],
        out_specs=pl.BlockSpec((_BE, 1), lambda i: (i, 0)),
        out_shape=jax.ShapeDtypeStruct((_E, 1), jnp.float32),
    )(m, Wm2, bm2r)


def kernel(x, edge_index, W1, b1, gamma, beta, W2, b2, Wm1, bm1, Wm2, bm2):
    src16 = edge_index[0].reshape(_NS, _E // (_NS * _CH), _CH)
    dst16 = edge_index[1].reshape(_NS, _E // (_NS * _CH), _CH)
    src32 = edge_index[0].reshape(_NW, _E // (_NW * _CH), _CH)
    dst32 = edge_index[1].reshape(_NW, _E // (_NW * _CH), _CH)

    partials = _sc_degree(dst32)
    dinv_col = _tc_dinv(partials).reshape(_N, 1)
    gA, gB = _tc_layer1(x, W1, b1.reshape(1, _H), dinv_col)
    aggA, aggB = _sc_aggregate(gA, gB, src16, dst16)
    h1, s1, s2 = _tc_mid_a(aggA, aggB, gA, gB, dinv_col)
    g2A, g2B = _tc_mid_b(h1, s1, s2, gamma.reshape(1, _H), beta.reshape(1, _H),
                         W2, b2.reshape(1, _H), dinv_col)
    agg2A, agg2B = _sc_aggregate(g2A, g2B, src16, dst16)
    P1, P2 = _tc_final_nodes(agg2A, agg2B, g2A, g2B, dinv_col,
                             Wm1, bm1.reshape(1, _H))
    m = _sc_edge_sum(P1, P2, src32, dst32)
    return _tc_classifier(m, Wm2, bm2.reshape(1, 1))
